# baseline (speedup 1.0000x reference)
import functools

import jax
import jax.numpy as jnp
from jax import lax
from jax.experimental import pallas as pl
from jax.experimental.pallas import tpu as pltpu
from jax.experimental.pallas import tpu_sc as plsc

_LANE = 16
_BLK = 128


def _relayout_table(table_t, tail_rows, *, vocab, embed, num_cores, num_subcores):
    nw = num_cores * num_subcores
    full_cols = vocab // _BLK
    tail = vocab - full_cols * _BLK
    base_n = full_cols // nw
    extra = full_cols - base_n * nw

    mesh = plsc.VectorSubcoreMesh(core_axis_name="c", subcore_axis_name="s")

    @functools.partial(
        pl.kernel,
        mesh=mesh,
        compiler_params=pltpu.CompilerParams(use_tc_tiling_on_sc=True, needs_layout_passes=False),
        out_type=jax.ShapeDtypeStruct((vocab, _BLK), jnp.float32),
        scratch_types=[
            pltpu.VMEM((embed, _BLK), jnp.float32),
            pltpu.VMEM((_BLK, _BLK), jnp.float32),
        ],
    )
    def k(tab_hbm, tail_hbm, pad_hbm, slab_v, rows_v):
        wid = lax.axis_index("s") * num_cores + lax.axis_index("c")
        n_w = jnp.where(wid < extra, base_n + 1, base_n)
        start = wid * base_n + jnp.minimum(wid, extra)

        iotas = [16 * k0 + lax.iota(jnp.int32, _LANE) for k0 in range(_BLK // _LANE)]

        def do_block(c):
            pltpu.sync_copy(tab_hbm.at[:, pl.ds(c * _BLK, _BLK)], slab_v)
            for e in range(embed):
                col = jnp.full((_LANE,), e, jnp.int32)
                for k0 in range(_BLK // _LANE):
                    v = slab_v[e, pl.ds(16 * k0, _LANE)]
                    plsc.store_scatter(rows_v, [iotas[k0], col], v)
            pltpu.sync_copy(rows_v, pad_hbm.at[pl.ds(c * _BLK, _BLK)])

        def body(i, _):
            @pl.when(i < n_w)
            def _():
                do_block(start + i)
            return 0

        lax.fori_loop(0, base_n + (1 if extra else 0), body, 0)

        if tail:
            @pl.when(wid == nw - 1)
            def _():
                pltpu.sync_copy(tail_hbm, rows_v.at[pl.ds(0, tail)])
                pltpu.sync_copy(
                    rows_v.at[pl.ds(0, tail)],
                    pad_hbm.at[pl.ds(full_cols * _BLK, tail)],
                )

    return k(table_t, tail_rows)


def _gather_native(input_t, pad_tab, *, batch, hist, embed, vocab,
                   num_cores, num_subcores):
    nw = num_cores * num_subcores
    blocks = batch // _BLK
    bpw = blocks // nw

    mesh = plsc.VectorSubcoreMesh(core_axis_name="c", subcore_axis_name="s")

    @functools.partial(
        pl.kernel,
        mesh=mesh,
        compiler_params=pltpu.CompilerParams(use_tc_tiling_on_sc=True, needs_layout_passes=False),
        out_type=jax.ShapeDtypeStruct((hist, embed, batch), jnp.float32),
        scratch_types=[
            pltpu.VMEM((_BLK,), jnp.int32),
            pltpu.VMEM((_BLK, _BLK), jnp.float32),
            pltpu.VMEM((embed, _BLK), jnp.float32),
            pltpu.SemaphoreType.DMA,
        ],
    )
    def k(idx_hbm, tab_hbm, out_hbm, idx_v, slab_v, oslab_v, sem):
        wid = lax.axis_index("s") * num_cores + lax.axis_index("c")

        e_iotas = [16 * k0 + lax.iota(jnp.int32, _LANE) for k0 in range(embed // _LANE)]

        def do_slab(h, bb):
            b0 = bb * _BLK
            pltpu.sync_copy(idx_hbm.at[h, pl.ds(b0, _BLK)], idx_v)
            pltpu.async_copy(tab_hbm.at[idx_v], slab_v, sem).wait()

            def tr(j, _):
                colj = jnp.full((_LANE,), j, jnp.int32)
                for k0 in range(embed // _LANE):
                    v = slab_v[j, pl.ds(16 * k0, _LANE)]
                    plsc.store_scatter(oslab_v, [e_iotas[k0], colj], v)
                return 0

            lax.fori_loop(0, _BLK, tr, 0)
            pltpu.sync_copy(oslab_v, out_hbm.at[h, :, pl.ds(b0, _BLK)])

        def body(t, _):
            h = t // bpw
            bb = wid * bpw + (t % bpw)
            do_slab(h, bb)
            return 0

        lax.fori_loop(0, hist * bpw, body, 0)

    return k(input_t, pad_tab)


def kernel(input, table):
    batch, hist = input.shape
    vocab, embed = table.shape
    info = plsc.get_sparse_core_info()
    nc, ns = info.num_cores, info.num_subcores

    table_t = table.T
    input_t = input.T.astype(jnp.int32)

    tail = vocab % _BLK
    tail_rows = jnp.pad(
        lax.slice(table, (vocab - tail, 0), (vocab, embed)),
        ((0, 0), (0, _BLK - embed)),
    )
    pad_tab = _relayout_table(
        table_t, tail_rows, vocab=vocab, embed=embed, num_cores=nc, num_subcores=ns
    )
    out = _gather_native(
        input_t, pad_tab, batch=batch, hist=hist, embed=embed, vocab=vocab,
        num_cores=nc, num_subcores=ns,
    )
    return out.transpose(2, 0, 1)


# speedup vs baseline: 1.4874x; 1.4874x over previous
import functools

import jax
import jax.numpy as jnp
from jax import lax
from jax.experimental import pallas as pl
from jax.experimental.pallas import tpu as pltpu
from jax.experimental.pallas import tpu_sc as plsc

_LANE = 16
_BLK = 128


def _relayout_table(table_t, tail_rows, *, vocab, embed, num_cores, num_subcores):
    nw = num_cores * num_subcores
    full_cols = vocab // _BLK
    tail = vocab - full_cols * _BLK
    base_n = full_cols // nw
    extra = full_cols - base_n * nw

    mesh = plsc.VectorSubcoreMesh(core_axis_name="c", subcore_axis_name="s")

    @functools.partial(
        pl.kernel,
        mesh=mesh,
        compiler_params=pltpu.CompilerParams(use_tc_tiling_on_sc=True, needs_layout_passes=False),
        out_type=jax.ShapeDtypeStruct((vocab, _BLK), jnp.float32),
        scratch_types=[
            pltpu.VMEM((embed, _BLK), jnp.float32),
            pltpu.VMEM((_BLK, _BLK), jnp.float32),
        ],
    )
    def k(tab_hbm, tail_hbm, pad_hbm, slab_v, rows_v):
        wid = lax.axis_index("s") * num_cores + lax.axis_index("c")
        n_w = jnp.where(wid < extra, base_n + 1, base_n)
        start = wid * base_n + jnp.minimum(wid, extra)

        iotas = [16 * k0 + lax.iota(jnp.int32, _LANE) for k0 in range(_BLK // _LANE)]

        def do_block(c):
            pltpu.sync_copy(tab_hbm.at[:, pl.ds(c * _BLK, _BLK)], slab_v)
            for e in range(embed):
                for k0 in range(_BLK // _LANE):
                    col = (iotas[k0] + e) & (embed - 1)
                    v = slab_v[e, pl.ds(16 * k0, _LANE)]
                    plsc.store_scatter(rows_v, [iotas[k0], col], v)
            pltpu.sync_copy(rows_v, pad_hbm.at[pl.ds(c * _BLK, _BLK)])

        def body(i, _):
            @pl.when(i < n_w)
            def _():
                do_block(start + i)
            return 0

        lax.fori_loop(0, base_n + (1 if extra else 0), body, 0)

        if tail:
            @pl.when(wid == nw - 1)
            def _():
                pltpu.sync_copy(tail_hbm, rows_v.at[pl.ds(0, tail)])
                pltpu.sync_copy(
                    rows_v.at[pl.ds(0, tail)],
                    pad_hbm.at[pl.ds(full_cols * _BLK, tail)],
                )

    return k(table_t, tail_rows)


def _gather_native(input_t, pad_tab, *, batch, hist, embed, vocab,
                   num_cores, num_subcores):
    nw = num_cores * num_subcores
    blocks = batch // _BLK
    bpw = blocks // nw

    mesh = plsc.VectorSubcoreMesh(core_axis_name="c", subcore_axis_name="s")

    @functools.partial(
        pl.kernel,
        mesh=mesh,
        compiler_params=pltpu.CompilerParams(use_tc_tiling_on_sc=True, needs_layout_passes=False),
        out_type=jax.ShapeDtypeStruct((hist, embed, batch), jnp.float32),
        scratch_types=[
            pltpu.VMEM((_BLK,), jnp.int32),
            pltpu.VMEM((_BLK, _BLK), jnp.float32),
            pltpu.VMEM((embed, _BLK), jnp.float32),
            pltpu.SemaphoreType.DMA,
        ],
    )
    def k(idx_hbm, tab_hbm, out_hbm, idx_v, slab_v, oslab_v, sem):
        wid = lax.axis_index("s") * num_cores + lax.axis_index("c")

        j_iotas = [16 * k0 + lax.iota(jnp.int32, _LANE) for k0 in range(_BLK // _LANE)]

        def do_slab(h, bb):
            b0 = bb * _BLK
            pltpu.sync_copy(idx_hbm.at[h, pl.ds(b0, _BLK)], idx_v)
            pltpu.async_copy(tab_hbm.at[idx_v], slab_v, sem).wait()

            for k0 in range(_BLK // _LANE):
                key = idx_v[pl.ds(16 * k0, _LANE)] & (_BLK - 1)
                for e in range(embed):
                    col = (key + e) & (embed - 1)
                    v = plsc.load_gather(slab_v, [j_iotas[k0], col])
                    oslab_v[e, pl.ds(16 * k0, _LANE)] = v

            pltpu.sync_copy(oslab_v, out_hbm.at[h, :, pl.ds(b0, _BLK)])

        def body(t, _):
            h = t // bpw
            bb = wid * bpw + (t % bpw)
            do_slab(h, bb)
            return 0

        lax.fori_loop(0, hist * bpw, body, 0)

    return k(input_t, pad_tab)


def kernel(input, table):
    batch, hist = input.shape
    vocab, embed = table.shape
    info = plsc.get_sparse_core_info()
    nc, ns = info.num_cores, info.num_subcores

    table_t = table.T
    input_t = input.T.astype(jnp.int32)

    tail = vocab % _BLK
    r = jnp.arange(tail)[:, None]
    c = jnp.arange(_BLK)[None, :]
    feat = (c - r) & (embed - 1)
    tail_rows = table[(vocab - tail) + r, feat]
    pad_tab = _relayout_table(
        table_t, tail_rows, vocab=vocab, embed=embed, num_cores=nc, num_subcores=ns
    )
    out = _gather_native(
        input_t, pad_tab, batch=batch, hist=hist, embed=embed, vocab=vocab,
        num_cores=nc, num_subcores=ns,
    )
    return out.transpose(2, 0, 1)


# speedup vs baseline: 1.7011x; 1.1437x over previous
import functools

import jax
import jax.numpy as jnp
from jax import lax
from jax.experimental import pallas as pl
from jax.experimental.pallas import tpu as pltpu
from jax.experimental.pallas import tpu_sc as plsc

_LANE = 16
_BLK = 128
_AC = 256
_BR = 256


def _relayout_table(table_t, tail_rows, *, vocab, embed, num_cores, num_subcores):
    nw = num_cores * num_subcores
    full_cols = vocab // _BLK
    tail = vocab - full_cols * _BLK
    n_groups = full_cols * _BLK // _AC
    base_n = n_groups // nw
    extra = n_groups - base_n * nw

    mesh = plsc.VectorSubcoreMesh(core_axis_name="c", subcore_axis_name="s")

    @functools.partial(
        pl.kernel,
        mesh=mesh,
        compiler_params=pltpu.CompilerParams(use_tc_tiling_on_sc=True, needs_layout_passes=False),
        out_type=jax.ShapeDtypeStruct((vocab, _BLK), jnp.float32),
        scratch_types=[
            pltpu.VMEM((embed, _AC), jnp.float32),
            pltpu.VMEM((_AC, _BLK), jnp.float32),
        ],
    )
    def k(tab_hbm, tail_hbm, pad_hbm, slab_v, rows_v):
        wid = lax.axis_index("s") * num_cores + lax.axis_index("c")
        n_w = jnp.where(wid < extra, base_n + 1, base_n)
        start = wid * base_n + jnp.minimum(wid, extra)

        iotas = [16 * k0 + lax.iota(jnp.int32, _LANE) for k0 in range(_AC // _LANE)]

        def do_block(g):
            pltpu.sync_copy(tab_hbm.at[:, pl.ds(g * _AC, _AC)], slab_v)
            for e in range(embed):
                for k0 in range(_AC // _LANE):
                    col = (iotas[k0] + e) & (embed - 1)
                    v = slab_v[e, pl.ds(16 * k0, _LANE)]
                    plsc.store_scatter(rows_v, [iotas[k0], col], v)
            pltpu.sync_copy(rows_v, pad_hbm.at[pl.ds(g * _AC, _AC)])

        def body(i, _):
            @pl.when(i < n_w)
            def _():
                do_block(start + i)
            return 0

        lax.fori_loop(0, base_n + (1 if extra else 0), body, 0)

        if tail:
            @pl.when(wid == nw - 1)
            def _():
                pltpu.sync_copy(tail_hbm, rows_v.at[pl.ds(0, tail)])
                pltpu.sync_copy(
                    rows_v.at[pl.ds(0, tail)],
                    pad_hbm.at[pl.ds(full_cols * _BLK, tail)],
                )

    return k(table_t, tail_rows)


def _gather_native(input_t, pad_tab, *, batch, hist, embed, vocab,
                   num_cores, num_subcores):
    nw = num_cores * num_subcores
    b_per_w = batch // nw
    chunks_per_h = b_per_w // _BR

    mesh = plsc.VectorSubcoreMesh(core_axis_name="c", subcore_axis_name="s")

    @functools.partial(
        pl.kernel,
        mesh=mesh,
        compiler_params=pltpu.CompilerParams(use_tc_tiling_on_sc=True, needs_layout_passes=False),
        out_type=jax.ShapeDtypeStruct((hist, embed, batch), jnp.float32),
        scratch_types=[
            pltpu.VMEM((_BR,), jnp.int32),
            pltpu.VMEM((_BR, _BLK), jnp.float32),
            pltpu.VMEM((embed, _BR), jnp.float32),
            pltpu.SemaphoreType.DMA,
        ],
    )
    def k(idx_hbm, tab_hbm, out_hbm, idx_v, slab_v, oslab_v, sem):
        wid = lax.axis_index("s") * num_cores + lax.axis_index("c")

        j_iotas = [16 * k0 + lax.iota(jnp.int32, _LANE) for k0 in range(_BR // _LANE)]

        def do_slab(h, ch):
            b0 = wid * b_per_w + ch * _BR
            pltpu.sync_copy(idx_hbm.at[h, pl.ds(b0, _BR)], idx_v)
            pltpu.async_copy(tab_hbm.at[idx_v], slab_v, sem).wait()

            for k0 in range(_BR // _LANE):
                key = idx_v[pl.ds(16 * k0, _LANE)] & (_BLK - 1)
                for e in range(embed):
                    col = (key + e) & (embed - 1)
                    v = plsc.load_gather(slab_v, [j_iotas[k0], col])
                    oslab_v[e, pl.ds(16 * k0, _LANE)] = v

            pltpu.sync_copy(oslab_v, out_hbm.at[h, :, pl.ds(b0, _BR)])

        def body(t, _):
            do_slab(t // chunks_per_h, t % chunks_per_h)
            return 0

        lax.fori_loop(0, hist * chunks_per_h, body, 0)

    return k(input_t, pad_tab)


def kernel(input, table):
    batch, hist = input.shape
    vocab, embed = table.shape
    info = plsc.get_sparse_core_info()
    nc, ns = info.num_cores, info.num_subcores

    table_t = table.T
    input_t = input.T.astype(jnp.int32)

    tail = vocab % _BLK
    r = jnp.arange(tail)[:, None]
    c = jnp.arange(_BLK)[None, :]
    feat = (c - r) & (embed - 1)
    tail_rows = table[(vocab - tail) + r, feat]
    pad_tab = _relayout_table(
        table_t, tail_rows, vocab=vocab, embed=embed, num_cores=nc, num_subcores=ns
    )
    out = _gather_native(
        input_t, pad_tab, batch=batch, hist=hist, embed=embed, vocab=vocab,
        num_cores=nc, num_subcores=ns,
    )
    return out.transpose(2, 0, 1)


# speedup vs baseline: 1.7572x; 1.0329x over previous
import functools

import jax
import jax.numpy as jnp
from jax import lax
from jax.experimental import pallas as pl
from jax.experimental.pallas import tpu as pltpu
from jax.experimental.pallas import tpu_sc as plsc

_LANE = 16
_BLK = 128
_AC = 256
_BR = 512


def _relayout_table(table_t, tail_rows, *, vocab, embed, num_cores, num_subcores):
    nw = num_cores * num_subcores
    full_cols = vocab // _BLK
    tail = vocab - full_cols * _BLK
    n_groups = full_cols * _BLK // _AC
    base_n = n_groups // nw
    extra = n_groups - base_n * nw

    mesh = plsc.VectorSubcoreMesh(core_axis_name="c", subcore_axis_name="s")

    @functools.partial(
        pl.kernel,
        mesh=mesh,
        compiler_params=pltpu.CompilerParams(use_tc_tiling_on_sc=True, needs_layout_passes=False),
        out_type=jax.ShapeDtypeStruct((vocab, _BLK), jnp.float32),
        scratch_types=[
            pltpu.VMEM((embed, _AC), jnp.float32),
            pltpu.VMEM((_AC, _BLK), jnp.float32),
        ],
    )
    def k(tab_hbm, tail_hbm, pad_hbm, slab_v, rows_v):
        wid = lax.axis_index("s") * num_cores + lax.axis_index("c")
        n_w = jnp.where(wid < extra, base_n + 1, base_n)
        start = wid * base_n + jnp.minimum(wid, extra)

        iotas = [16 * k0 + lax.iota(jnp.int32, _LANE) for k0 in range(_AC // _LANE)]

        def do_block(g):
            pltpu.sync_copy(tab_hbm.at[:, pl.ds(g * _AC, _AC)], slab_v)
            for e in range(embed):
                for k0 in range(_AC // _LANE):
                    col = (iotas[k0] + e) & (embed - 1)
                    v = slab_v[e, pl.ds(16 * k0, _LANE)]
                    plsc.store_scatter(rows_v, [iotas[k0], col], v)
            pltpu.sync_copy(rows_v, pad_hbm.at[pl.ds(g * _AC, _AC)])

        def body(i, _):
            @pl.when(i < n_w)
            def _():
                do_block(start + i)
            return 0

        lax.fori_loop(0, base_n + (1 if extra else 0), body, 0)

        if tail:
            @pl.when(wid == nw - 1)
            def _():
                pltpu.sync_copy(tail_hbm, rows_v.at[pl.ds(0, tail)])
                pltpu.sync_copy(
                    rows_v.at[pl.ds(0, tail)],
                    pad_hbm.at[pl.ds(full_cols * _BLK, tail)],
                )

    return k(table_t, tail_rows)


def _gather_native(input_t, pad_tab, *, batch, hist, embed, vocab,
                   num_cores, num_subcores):
    nw = num_cores * num_subcores
    b_per_w = batch // nw
    chunks_per_h = b_per_w // _BR

    mesh = plsc.VectorSubcoreMesh(core_axis_name="c", subcore_axis_name="s")

    @functools.partial(
        pl.kernel,
        mesh=mesh,
        compiler_params=pltpu.CompilerParams(use_tc_tiling_on_sc=True, needs_layout_passes=False),
        out_type=jax.ShapeDtypeStruct((hist, embed, batch), jnp.float32),
        scratch_types=[
            pltpu.VMEM((_BR,), jnp.int32),
            pltpu.VMEM((_BR, _BLK), jnp.float32),
            pltpu.VMEM((embed, _BR), jnp.float32),
            pltpu.SemaphoreType.DMA,
        ],
    )
    def k(idx_hbm, tab_hbm, out_hbm, idx_v, slab_v, oslab_v, sem):
        wid = lax.axis_index("s") * num_cores + lax.axis_index("c")

        j_iotas = [16 * k0 + lax.iota(jnp.int32, _LANE) for k0 in range(_BR // _LANE)]

        def do_slab(h, ch):
            b0 = wid * b_per_w + ch * _BR
            pltpu.sync_copy(idx_hbm.at[h, pl.ds(b0, _BR)], idx_v)
            pltpu.async_copy(tab_hbm.at[idx_v], slab_v, sem).wait()

            for k0 in range(_BR // _LANE):
                key = idx_v[pl.ds(16 * k0, _LANE)] & (_BLK - 1)
                for e in range(embed):
                    col = (key + e) & (embed - 1)
                    v = plsc.load_gather(slab_v, [j_iotas[k0], col])
                    oslab_v[e, pl.ds(16 * k0, _LANE)] = v

            pltpu.sync_copy(oslab_v, out_hbm.at[h, :, pl.ds(b0, _BR)])

        def body(t, _):
            do_slab(t // chunks_per_h, t % chunks_per_h)
            return 0

        lax.fori_loop(0, hist * chunks_per_h, body, 0)

    return k(input_t, pad_tab)


def kernel(input, table):
    batch, hist = input.shape
    vocab, embed = table.shape
    info = plsc.get_sparse_core_info()
    nc, ns = info.num_cores, info.num_subcores

    table_t = table.T
    input_t = input.T.astype(jnp.int32)

    tail = vocab % _BLK
    r = jnp.arange(tail)[:, None]
    c = jnp.arange(_BLK)[None, :]
    feat = (c - r) & (embed - 1)
    tail_rows = table[(vocab - tail) + r, feat]
    pad_tab = _relayout_table(
        table_t, tail_rows, vocab=vocab, embed=embed, num_cores=nc, num_subcores=ns
    )
    out = _gather_native(
        input_t, pad_tab, batch=batch, hist=hist, embed=embed, vocab=vocab,
        num_cores=nc, num_subcores=ns,
    )
    return out.transpose(2, 0, 1)
